# baseline (speedup 1.0000x reference)
import functools

import jax
import jax.numpy as jnp
from jax import lax
from jax.experimental import pallas as pl
from jax.experimental.pallas import tpu as pltpu
from jax.experimental.pallas import tpu_sc as plsc

_LANES = 16
_VOCAB_BLK = 25600


def _scores_body(w_ref, b_ref, t_ref, o_ref, *, inv_hist):
    t = t_ref[...]
    w = w_ref[...]
    s = lax.dot_general(w, t, (((1,), (0,)), ((), ())),
                        preferred_element_type=jnp.float32)
    o_ref[...] = (s[0] + b_ref[0]) * inv_hist


def _scores(table_t, W, b, hist):
    D, V = table_t.shape
    return pl.pallas_call(
        functools.partial(_scores_body, inv_hist=1.0 / float(hist)),
        in_specs=[
            pl.BlockSpec(memory_space=pltpu.VMEM),
            pl.BlockSpec(memory_space=pltpu.SMEM),
            pl.BlockSpec(memory_space=pltpu.VMEM),
        ],
        out_specs=pl.BlockSpec(memory_space=pltpu.VMEM),
        out_shape=jax.ShapeDtypeStruct((V,), jnp.float32),
    )(W, b, table_t)


def _round4(y):
    t = y * 10000.0
    t = (t + 8388608.0) - 8388608.0
    return t / 10000.0


def _pool(x_T, s_flat, batch, hist):
    info = plsc.get_sparse_core_info()
    nc, ns = info.num_cores, info.num_subcores
    nw = nc * ns
    b_per = batch // nw
    groups = b_per // _LANES
    len_s = s_flat.shape[0]

    mesh = plsc.VectorSubcoreMesh(core_axis_name="c", subcore_axis_name="s")

    @functools.partial(
        pl.kernel,
        out_type=jax.ShapeDtypeStruct((batch,), jnp.float32),
        mesh=mesh,
        scratch_types=[
            pltpu.VMEM((hist, b_per), jnp.int32),
            pltpu.VMEM((hist * b_per,), jnp.float32),
            pltpu.VMEM((b_per,), jnp.float32),
            pltpu.VMEM_SHARED((len_s,), jnp.float32),
            pltpu.SemaphoreType.DMA,
        ],
    )
    def run(xT_hbm, s_hbm, out_hbm, idx_t, vals_v, y_v, s_sh, sem):
        sid = lax.axis_index("s")
        wid = sid * nc + lax.axis_index("c")
        base = wid * b_per

        with jax.named_scope("s_stage"):
            @pl.when(sid == 0)
            def _():
                pltpu.sync_copy(s_hbm, s_sh)

        with jax.named_scope("idx_stage"):
            pltpu.sync_copy(xT_hbm.at[:, pl.ds(base, b_per)], idx_t)
        with jax.named_scope("barrier"):
            plsc.subcore_barrier()

        def fire(l, carry):
            pltpu.async_copy(
                s_sh.at[idx_t.at[l]],
                vals_v.at[pl.ds(l * b_per, b_per)],
                sem,
            )
            return carry

        with jax.named_scope("fire"):
            lax.fori_loop(0, hist, fire, 0)
        with jax.named_scope("drain"):
            pltpu.make_async_copy(
                s_hbm.at[pl.ds(0, hist * b_per)], vals_v, sem
            ).wait()

        zero = jnp.zeros((_LANES,), jnp.float32)

        def acc_body(l, accs):
            return tuple(
                accs[j] + vals_v[pl.ds(l * b_per + j * _LANES, _LANES)]
                for j in range(groups)
            )

        with jax.named_scope("accum"):
            accs = lax.fori_loop(0, hist, acc_body, (zero,) * groups)
            for j in range(groups):
                y = 1.0 / (1.0 + jnp.exp(-accs[j]))
                y_v[pl.ds(j * _LANES, _LANES)] = _round4(y)
        with jax.named_scope("out"):
            pltpu.sync_copy(y_v, out_hbm.at[pl.ds(wid * b_per, b_per)])

    return run(x_T, s_flat)


def kernel(x, emb_table, W, b):
    batch, hist = x.shape
    V, _ = emb_table.shape
    scores = _scores(emb_table.T, W, b, hist)
    x_T = x.T.astype(jnp.int32)
    out = _pool(x_T, scores.reshape(V), batch, hist)
    return out.reshape(batch, 1)


# speedup vs baseline: 1.0242x; 1.0242x over previous
import functools

import jax
import jax.numpy as jnp
from jax import lax
from jax.experimental import pallas as pl
from jax.experimental.pallas import tpu as pltpu
from jax.experimental.pallas import tpu_sc as plsc

_LANES = 16
_VOCAB_BLK = 25600


def _scores_body(w_ref, b_ref, t_ref, o_ref, *, inv_hist):
    t = t_ref[...]
    w = w_ref[...]
    s = lax.dot_general(w, t, (((1,), (0,)), ((), ())),
                        preferred_element_type=jnp.float32)
    o_ref[...] = (s[0] + b_ref[0]) * inv_hist


def _scores(table_t, W, b, hist):
    D, V = table_t.shape
    return pl.pallas_call(
        functools.partial(_scores_body, inv_hist=1.0 / float(hist)),
        in_specs=[
            pl.BlockSpec(memory_space=pltpu.VMEM),
            pl.BlockSpec(memory_space=pltpu.SMEM),
            pl.BlockSpec(memory_space=pltpu.VMEM),
        ],
        out_specs=pl.BlockSpec(memory_space=pltpu.VMEM),
        out_shape=jax.ShapeDtypeStruct((V,), jnp.float32),
    )(W, b, table_t)


def _round4(y):
    t = y * 10000.0
    t = (t + 8388608.0) - 8388608.0
    return t / 10000.0


def _pool(x_T, s_flat, batch, hist):
    info = plsc.get_sparse_core_info()
    nc, ns = info.num_cores, info.num_subcores
    nw = nc * ns
    b_per = batch // nw
    groups = b_per // _LANES
    len_s = s_flat.shape[0]
    slab = (-(len_s // -ns) + 7) & ~7
    tail = len_s - slab * (ns - 1)

    mesh = plsc.VectorSubcoreMesh(core_axis_name="c", subcore_axis_name="s")

    @functools.partial(
        pl.kernel,
        out_type=jax.ShapeDtypeStruct((batch,), jnp.float32),
        mesh=mesh,
        scratch_types=[
            pltpu.VMEM((hist, b_per), jnp.int32),
            pltpu.VMEM((hist * b_per,), jnp.float32),
            pltpu.VMEM((b_per,), jnp.float32),
            pltpu.VMEM_SHARED((len_s,), jnp.float32),
            pltpu.VMEM((slab,), jnp.float32),
            pltpu.SemaphoreType.DMA,
            pltpu.SemaphoreType.DMA,
        ],
    )
    def run(xT_hbm, s_hbm, out_hbm, idx_t, vals_v, y_v, s_sh, s_vm, sem, sem_s):
        sid = lax.axis_index("s")
        wid = sid * nc + lax.axis_index("c")
        base = wid * b_per

        off = sid * slab
        with jax.named_scope("s_stage"):
            @pl.when(sid < ns - 1)
            def _():
                pltpu.async_copy(s_hbm.at[pl.ds(off, slab)], s_vm, sem_s)

            @pl.when(sid == ns - 1)
            def _():
                pltpu.async_copy(s_hbm.at[pl.ds(off, tail)],
                                 s_vm.at[pl.ds(0, tail)], sem_s)

        with jax.named_scope("idx_stage"):
            pltpu.sync_copy(xT_hbm.at[:, pl.ds(base, b_per)], idx_t)

        with jax.named_scope("s_wait"):
            @pl.when(sid < ns - 1)
            def _():
                pltpu.make_async_copy(
                    s_hbm.at[pl.ds(off, slab)], s_vm, sem_s).wait()
                pltpu.sync_copy(s_vm, s_sh.at[pl.ds(off, slab)])

            @pl.when(sid == ns - 1)
            def _():
                pltpu.make_async_copy(
                    s_hbm.at[pl.ds(off, tail)],
                    s_vm.at[pl.ds(0, tail)], sem_s).wait()
                pltpu.sync_copy(s_vm.at[pl.ds(0, tail)],
                                s_sh.at[pl.ds(off, tail)])

        with jax.named_scope("barrier"):
            plsc.subcore_barrier()

        def fire(l, carry):
            pltpu.async_copy(
                s_sh.at[idx_t.at[l]],
                vals_v.at[pl.ds(l * b_per, b_per)],
                sem,
            )
            return carry

        with jax.named_scope("fire"):
            lax.fori_loop(0, hist, fire, 0)
        with jax.named_scope("drain"):
            pltpu.make_async_copy(
                s_hbm.at[pl.ds(0, hist * b_per)], vals_v, sem
            ).wait()

        zero = jnp.zeros((_LANES,), jnp.float32)

        def acc_body(l, accs):
            return tuple(
                accs[j] + vals_v[pl.ds(l * b_per + j * _LANES, _LANES)]
                for j in range(groups)
            )

        with jax.named_scope("accum"):
            accs = lax.fori_loop(0, hist, acc_body, (zero,) * groups)
            for j in range(groups):
                y = 1.0 / (1.0 + jnp.exp(-accs[j]))
                y_v[pl.ds(j * _LANES, _LANES)] = _round4(y)
        with jax.named_scope("out"):
            pltpu.sync_copy(y_v, out_hbm.at[pl.ds(wid * b_per, b_per)])

    return run(x_T, s_flat)


def kernel(x, emb_table, W, b):
    batch, hist = x.shape
    V, _ = emb_table.shape
    scores = _scores(emb_table.T, W, b, hist)
    x_T = x.T.astype(jnp.int32)
    out = _pool(x_T, scores.reshape(V), batch, hist)
    return out.reshape(batch, 1)
